# Initial kernel scaffold; baseline (speedup 1.0000x reference)
import jax
import jax.numpy as jnp
from jax.experimental import pallas as pl


def kernel(x, edge_index, edge_type, pairs, W1, root1, b1, W2, root2, b2, dec_w1, dec_b1, dec_w2, dec_b2):
    raise NotImplementedError("write your pallas kernel here")


# speedup vs baseline: 23.9230x; 23.9230x over previous
import jax
import jax.numpy as jnp
from jax import lax
from jax.experimental import pallas as pl
from jax.experimental.pallas import tpu as pltpu
from jax.experimental.pallas import tpu_sc as plsc

N = 10000
E = 320000
R = 8
F_IN = 128
HID = 256
F_OUT = 128
NCLS = 16
P = 100000
P_PAD = 102400

NC = 2
NS = 16


BE = 200


def _make_sc_seg():
    ept = E // (NC * NS)
    nb = ept // BE

    def body(table, gix, d32, out, gidx, sidx,
             rows, zbuf, destbuf, sem):
        core = lax.axis_index("c")
        tid = lax.axis_index("s")
        base_e = (core * NS + tid) * ept

        @pl.loop(0, 40)
        def _(i):
            for j in range(F_OUT // 16):
                zbuf[i, pl.ds(j * 16, 16)] = jnp.zeros((16,), jnp.float32)

        zrows = 1000

        @pl.when(tid < 10)
        def _():
            @pl.loop(0, zrows // 40)
            def _(k):
                pltpu.sync_copy(zbuf,
                                destbuf.at[pl.ds(tid * zrows + k * 40, 40)])

        plsc.subcore_barrier()

        @pl.loop(0, nb)
        def _(b):
            off = base_e + b * BE
            pltpu.sync_copy(d32.at[pl.ds(off, BE)], sidx)
            pltpu.sync_copy(gix.at[pl.ds(off, BE)], gidx)

            pltpu.async_copy(table.at[gidx], rows, sem).wait()
            pltpu.sync_copy(rows, destbuf.at[sidx], add=True)

        plsc.subcore_barrier()

        @pl.when(tid < 10)
        def _():
            pltpu.sync_copy(destbuf.at[pl.ds(tid * zrows, zrows)],
                            out.at[pl.ds(core * N + tid * zrows, zrows)])

        plsc.subcore_barrier()

    mesh = plsc.VectorSubcoreMesh(core_axis_name="c", subcore_axis_name="s")

    def run(table, gix, d32):
        return pl.kernel(
            body,
            out_type=jax.ShapeDtypeStruct((NC * N, F_OUT), jnp.float32),
            mesh=mesh,
            scratch_types=[
                pltpu.VMEM((BE,), jnp.int32),
                pltpu.VMEM((BE,), jnp.int32),
                pltpu.VMEM((BE, F_OUT), jnp.float32),
                pltpu.VMEM((40, F_OUT), jnp.float32),
                pltpu.VMEM_SHARED((N, F_OUT), jnp.float32),
                pltpu.SemaphoreType.DMA,
            ],
        )(table, gix, d32)

    return run


_sc_seg = _make_sc_seg()


PPT = P_PAD // (NC * NS)
B3 = 320
NB3 = PPT // B3


def _sc_dec_body(hAB, psrc, pdst, z, idxa, idxb, tmp, u, v, sem):
    core = lax.axis_index("c")
    tid = lax.axis_index("s")
    w = core * NS + tid
    base_p = w * PPT

    @pl.loop(0, NB3)
    def _(b):
        off = base_p + b * B3
        pltpu.sync_copy(psrc.at[pl.ds(off, B3)], idxa)
        pltpu.sync_copy(pdst.at[pl.ds(off, B3)], tmp)

        @pl.loop(0, B3 // 16)
        def _(i):
            s = pl.ds(i * 16, 16)
            idxb[s] = tmp[s] + N

        pltpu.async_copy(hAB.at[idxa], u, sem).wait()
        pltpu.async_copy(hAB.at[idxb], v, sem).wait()

        @pl.loop(0, B3)
        def _(i):
            for j in range(F_OUT // 16):
                s = pl.ds(j * 16, 16)
                u[i, s] = jnp.maximum(u[i, s] + v[i, s], 0.0)

        pltpu.sync_copy(u, z.at[pl.ds(off, B3)])


def _sc_dec(hAB, psrc, pdst):
    mesh = plsc.VectorSubcoreMesh(core_axis_name="c", subcore_axis_name="s")
    return pl.kernel(
        _sc_dec_body,
        out_type=jax.ShapeDtypeStruct((P_PAD, F_OUT), jnp.float32),
        mesh=mesh,
        scratch_types=[
            pltpu.VMEM((B3,), jnp.int32),
            pltpu.VMEM((B3,), jnp.int32),
            pltpu.VMEM((B3,), jnp.int32),
            pltpu.VMEM((B3, F_OUT), jnp.float32),
            pltpu.VMEM((B3, F_OUT), jnp.float32),
            pltpu.SemaphoreType.DMA,
        ],
    )(hAB, psrc, pdst)


NBLK = 1000
NNB = N // NBLK


def _tc_eidx_body(ei_blk, et_blk, gix_blk, d_blk):
    gix_blk[...] = et_blk[...] * N + ei_blk[0] + N
    d_blk[...] = ei_blk[1]


def _tc_eidx(edge_index2, edge_type2):
    eb = 2500
    return pl.pallas_call(
        _tc_eidx_body,
        grid=(E // (eb * 128),),
        in_specs=[
            pl.BlockSpec((2, eb, 128), lambda i: (0, i, 0)),
            pl.BlockSpec((eb, 128), lambda i: (i, 0)),
        ],
        out_specs=[
            pl.BlockSpec((eb, 128), lambda i: (i, 0)),
            pl.BlockSpec((eb, 128), lambda i: (i, 0)),
        ],
        out_shape=[
            jax.ShapeDtypeStruct((E // 128, 128), jnp.int32),
            jax.ShapeDtypeStruct((E // 128, 128), jnp.int32),
        ],
    )(edge_index2, edge_type2)


def _tc_transform_body(h_blk, w_blk, b_blk, out_blk):
    r = pl.program_id(1)
    y = jnp.dot(h_blk[...], w_blk[0], preferred_element_type=jnp.float32)
    out_blk[...] = jnp.where(r == 0, y + b_blk[...], y)


def _tc_transform(h, Wall, bias):
    in_dim = h.shape[1]
    return pl.pallas_call(
        _tc_transform_body,
        grid=(NNB, R + 1),
        in_specs=[
            pl.BlockSpec((NBLK, in_dim), lambda nb, r: (nb, 0)),
            pl.BlockSpec((1, in_dim, 128), lambda nb, r: (r, 0, 0)),
            pl.BlockSpec((1, 128), lambda nb, r: (0, 0)),
        ],
        out_specs=pl.BlockSpec((NBLK, 128), lambda nb, r: (r * NNB + nb, 0)),
        out_shape=jax.ShapeDtypeStruct(((R + 1) * N, 128), jnp.float32),
    )(h, Wall, bias)


def _tc_h1_body(lo_blk, hi_blk, pa0, pa1, pb0, pb1, out_blk):
    h = pl.program_id(1)
    s = jnp.where(h == 0,
                  lo_blk[...] + pa0[...] + pa1[...],
                  hi_blk[...] + pb0[...] + pb1[...])
    out_blk[...] = jnp.maximum(s, 0.0)


def _tc_h1(y1lo, y1hi, partsA, partsB):
    return pl.pallas_call(
        _tc_h1_body,
        grid=(NNB, 2),
        in_specs=[
            pl.BlockSpec((NBLK, 128), lambda nb, h: (nb, 0)),
            pl.BlockSpec((NBLK, 128), lambda nb, h: (nb, 0)),
            pl.BlockSpec((NBLK, 128), lambda nb, h: (nb, 0)),
            pl.BlockSpec((NBLK, 128), lambda nb, h: (NNB + nb, 0)),
            pl.BlockSpec((NBLK, 128), lambda nb, h: (nb, 0)),
            pl.BlockSpec((NBLK, 128), lambda nb, h: (NNB + nb, 0)),
        ],
        out_specs=pl.BlockSpec((NBLK, 128), lambda nb, h: (nb, h)),
        out_shape=jax.ShapeDtypeStruct((N, HID), jnp.float32),
    )(y1lo, y1hi, partsA, partsA, partsB, partsB)


def _tc_dec_prep_body(root_blk, p0_blk, p1_blk, w_blk, b_blk, out_blk):
    g = pl.program_id(0)
    h2 = root_blk[...] + p0_blk[...] + p1_blk[...]
    y = jnp.dot(h2, w_blk[...], preferred_element_type=jnp.float32)
    out_blk[...] = jnp.where(g == 0, y + b_blk[...], y)


def _tc_dec_prep(y2all, parts, dec_w1, dec_b1):
    return pl.pallas_call(
        _tc_dec_prep_body,
        grid=(2, NNB),
        in_specs=[
            pl.BlockSpec((NBLK, F_OUT), lambda g, nb: (nb, 0)),
            pl.BlockSpec((NBLK, F_OUT), lambda g, nb: (nb, 0)),
            pl.BlockSpec((NBLK, F_OUT), lambda g, nb: (NNB + nb, 0)),
            pl.BlockSpec((F_OUT, F_OUT), lambda g, nb: (g, 0)),
            pl.BlockSpec((1, F_OUT), lambda g, nb: (0, 0)),
        ],
        out_specs=pl.BlockSpec((NBLK, F_OUT), lambda g, nb: (g * NNB + nb, 0)),
        out_shape=jax.ShapeDtypeStruct((2 * N, F_OUT), jnp.float32),
    )(y2all, parts, parts, dec_w1, dec_b1)


ZBLK = 1600


def _tc_out_body(z_blk, w_blk, b_blk, out_blk):
    out_blk[...] = (
        jnp.dot(z_blk[...], w_blk[...], preferred_element_type=jnp.float32)
        + b_blk[...]
    )


def _tc_out(z, dec_w2, dec_b2):
    return pl.pallas_call(
        _tc_out_body,
        grid=(P_PAD // ZBLK,),
        in_specs=[
            pl.BlockSpec((ZBLK, F_OUT), lambda i: (i, 0)),
            pl.BlockSpec((F_OUT, NCLS), lambda i: (0, 0)),
            pl.BlockSpec((1, NCLS), lambda i: (0, 0)),
        ],
        out_specs=pl.BlockSpec((ZBLK, NCLS), lambda i: (i, 0)),
        out_shape=jax.ShapeDtypeStruct((P_PAD, NCLS), jnp.float32),
    )(z, dec_w2, dec_b2)


@jax.jit
def _run(x, edge_index, edge_type, pairs, W1, root1, b1, W2, root2, b2,
         dec_w1, dec_b1, dec_w2, dec_b2):
    ei2 = jnp.asarray(edge_index, jnp.int32).reshape(2, E // 128, 128)
    et2 = jnp.asarray(edge_type, jnp.int32).reshape(E // 128, 128)
    gix2, d2 = _tc_eidx(ei2, et2)
    gix = gix2.reshape(E)
    d32 = d2.reshape(E)
    pairs32 = jnp.asarray(pairs, jnp.int32)
    pairs_p = jnp.concatenate(
        [pairs32, jnp.zeros((P_PAD - P, 2), jnp.int32)], axis=0)
    psrc = pairs_p[:, 0]
    pdst = pairs_p[:, 1]

    root1r = root1.reshape(1, F_IN, HID)
    W1lo = jnp.concatenate([root1r[:, :, :128], W1[:, :, :128]], axis=0)
    W1hi = jnp.concatenate([root1r[:, :, 128:], W1[:, :, 128:]], axis=0)
    y1lo = _tc_transform(x, W1lo, b1[:128].reshape(1, 128))
    y1hi = _tc_transform(x, W1hi, b1[128:].reshape(1, 128))
    partsA = _sc_seg(y1lo, gix, d32)
    y1hi2, _ = lax.optimization_barrier((y1hi, partsA))
    partsB = _sc_seg(y1hi2, gix, d32)
    h1 = _tc_h1(y1lo, y1hi2, partsA, partsB)

    W2all = jnp.concatenate([root2.reshape(1, HID, F_OUT), W2], axis=0)
    y2all = _tc_transform(h1, W2all, b2.reshape(1, F_OUT))
    parts2 = _sc_seg(y2all, gix, d32)

    hAB = _tc_dec_prep(y2all, parts2, dec_w1, dec_b1.reshape(1, F_OUT))
    z = _sc_dec(hAB, psrc, pdst)
    logits = _tc_out(z, dec_w2, dec_b2.reshape(1, NCLS))
    return logits[:P]


def kernel(x, edge_index, edge_type, pairs, W1, root1, b1, W2, root2, b2,
           dec_w1, dec_b1, dec_w2, dec_b2):
    return _run(x, edge_index, edge_type, pairs, W1, root1, b1, W2, root2, b2,
                dec_w1, dec_b1, dec_w2, dec_b2)


# speedup vs baseline: 25.3881x; 1.0612x over previous
import jax
import jax.numpy as jnp
from jax import lax
from jax.experimental import pallas as pl
from jax.experimental.pallas import tpu as pltpu
from jax.experimental.pallas import tpu_sc as plsc

N = 10000
E = 320000
R = 8
F_IN = 128
HID = 256
F_OUT = 128
NCLS = 16
P = 100000
P_PAD = 102400

NC = 2
NS = 16


BE = 80


def _make_sc_seg():
    ept = E // (NC * NS)
    nb = ept // BE
    assert nb % 2 == 1 and nb >= 3

    def body(table, gix, d32, out, gidx0, gidx1, sidx0, sidx1,
             rows0, rows1, zbuf, destbuf, sem0, sem1):
        core = lax.axis_index("c")
        tid = lax.axis_index("s")
        base_e = (core * NS + tid) * ept

        @pl.loop(0, 40)
        def _(i):
            for j in range(F_OUT // 16):
                zbuf[i, pl.ds(j * 16, 16)] = jnp.zeros((16,), jnp.float32)

        zrows = 1000

        @pl.when(tid < 10)
        def _():
            @pl.loop(0, zrows // 40)
            def _(k):
                pltpu.sync_copy(zbuf,
                                destbuf.at[pl.ds(tid * zrows + k * 40, 40)])

        plsc.subcore_barrier()

        pltpu.sync_copy(d32.at[pl.ds(base_e, BE)], sidx0)
        pltpu.sync_copy(gix.at[pl.ds(base_e, BE)], gidx0)
        pltpu.async_copy(table.at[gidx0], rows0, sem0)

        @pl.loop(0, (nb - 1) // 2)
        def _(p):
            off1 = base_e + (2 * p + 1) * BE
            pltpu.sync_copy(d32.at[pl.ds(off1, BE)], sidx1)
            pltpu.sync_copy(gix.at[pl.ds(off1, BE)], gidx1)
            pltpu.async_copy(table.at[gidx1], rows1, sem1)

            pltpu.make_async_copy(table.at[gidx0], rows0, sem0).wait()
            pltpu.sync_copy(rows0, destbuf.at[sidx0], add=True)

            off0 = base_e + (2 * p + 2) * BE
            pltpu.sync_copy(d32.at[pl.ds(off0, BE)], sidx0)
            pltpu.sync_copy(gix.at[pl.ds(off0, BE)], gidx0)
            pltpu.async_copy(table.at[gidx0], rows0, sem0)

            pltpu.make_async_copy(table.at[gidx1], rows1, sem1).wait()
            pltpu.sync_copy(rows1, destbuf.at[sidx1], add=True)

        pltpu.make_async_copy(table.at[gidx0], rows0, sem0).wait()
        pltpu.sync_copy(rows0, destbuf.at[sidx0], add=True)

        plsc.subcore_barrier()

        @pl.when(tid < 10)
        def _():
            pltpu.sync_copy(destbuf.at[pl.ds(tid * zrows, zrows)],
                            out.at[pl.ds(core * N + tid * zrows, zrows)])

        plsc.subcore_barrier()

    mesh = plsc.VectorSubcoreMesh(core_axis_name="c", subcore_axis_name="s")

    def run(table, gix, d32):
        return pl.kernel(
            body,
            out_type=jax.ShapeDtypeStruct((NC * N, F_OUT), jnp.float32),
            mesh=mesh,
            scratch_types=[
                pltpu.VMEM((BE,), jnp.int32),
                pltpu.VMEM((BE,), jnp.int32),
                pltpu.VMEM((BE,), jnp.int32),
                pltpu.VMEM((BE,), jnp.int32),
                pltpu.VMEM((BE, F_OUT), jnp.float32),
                pltpu.VMEM((BE, F_OUT), jnp.float32),
                pltpu.VMEM((40, F_OUT), jnp.float32),
                pltpu.VMEM_SHARED((N, F_OUT), jnp.float32),
                pltpu.SemaphoreType.DMA,
                pltpu.SemaphoreType.DMA,
            ],
        )(table, gix, d32)

    return run


_sc_seg = _make_sc_seg()


PPT = P_PAD // (NC * NS)
B3 = 320
NB3 = PPT // B3


def _sc_dec_body(hAB, psrc, pdst, z, idxa, idxb, tmp, u, v, sem, sem2):
    core = lax.axis_index("c")
    tid = lax.axis_index("s")
    w = core * NS + tid
    base_p = w * PPT

    @pl.loop(0, NB3)
    def _(b):
        off = base_p + b * B3
        pltpu.sync_copy(psrc.at[pl.ds(off, B3)], idxa)
        pltpu.sync_copy(pdst.at[pl.ds(off, B3)], tmp)

        @pl.loop(0, B3 // 16)
        def _(i):
            s = pl.ds(i * 16, 16)
            idxb[s] = tmp[s] + N

        pltpu.async_copy(hAB.at[idxa], u, sem)
        pltpu.async_copy(hAB.at[idxb], v, sem2)
        pltpu.make_async_copy(hAB.at[idxa], u, sem).wait()
        pltpu.make_async_copy(hAB.at[idxb], v, sem2).wait()

        @pl.loop(0, B3, unroll=4)
        def _(i):
            for j in range(F_OUT // 16):
                s = pl.ds(j * 16, 16)
                u[i, s] = jnp.maximum(u[i, s] + v[i, s], 0.0)

        pltpu.sync_copy(u, z.at[pl.ds(off, B3)])


def _sc_dec(hAB, psrc, pdst):
    mesh = plsc.VectorSubcoreMesh(core_axis_name="c", subcore_axis_name="s")
    return pl.kernel(
        _sc_dec_body,
        out_type=jax.ShapeDtypeStruct((P_PAD, F_OUT), jnp.float32),
        mesh=mesh,
        scratch_types=[
            pltpu.VMEM((B3,), jnp.int32),
            pltpu.VMEM((B3,), jnp.int32),
            pltpu.VMEM((B3,), jnp.int32),
            pltpu.VMEM((B3, F_OUT), jnp.float32),
            pltpu.VMEM((B3, F_OUT), jnp.float32),
            pltpu.SemaphoreType.DMA,
            pltpu.SemaphoreType.DMA,
        ],
    )(hAB, psrc, pdst)


NBLK = 1000
NNB = N // NBLK


def _tc_eidx_body(ei_blk, et_blk, gix_blk, d_blk):
    gix_blk[...] = et_blk[...] * N + ei_blk[0] + N
    d_blk[...] = ei_blk[1]


def _tc_eidx(edge_index2, edge_type2):
    eb = 2500
    return pl.pallas_call(
        _tc_eidx_body,
        grid=(E // (eb * 128),),
        in_specs=[
            pl.BlockSpec((2, eb, 128), lambda i: (0, i, 0)),
            pl.BlockSpec((eb, 128), lambda i: (i, 0)),
        ],
        out_specs=[
            pl.BlockSpec((eb, 128), lambda i: (i, 0)),
            pl.BlockSpec((eb, 128), lambda i: (i, 0)),
        ],
        out_shape=[
            jax.ShapeDtypeStruct((E // 128, 128), jnp.int32),
            jax.ShapeDtypeStruct((E // 128, 128), jnp.int32),
        ],
    )(edge_index2, edge_type2)


def _tc_transform_body(h_blk, w_blk, b_blk, out_blk):
    r = pl.program_id(1)
    y = jnp.dot(h_blk[...], w_blk[0], preferred_element_type=jnp.float32)
    out_blk[...] = jnp.where(r == 0, y + b_blk[...], y)


def _tc_transform(h, Wall, bias):
    in_dim = h.shape[1]
    return pl.pallas_call(
        _tc_transform_body,
        grid=(NNB, R + 1),
        in_specs=[
            pl.BlockSpec((NBLK, in_dim), lambda nb, r: (nb, 0)),
            pl.BlockSpec((1, in_dim, 128), lambda nb, r: (r, 0, 0)),
            pl.BlockSpec((1, 128), lambda nb, r: (0, 0)),
        ],
        out_specs=pl.BlockSpec((NBLK, 128), lambda nb, r: (r * NNB + nb, 0)),
        out_shape=jax.ShapeDtypeStruct(((R + 1) * N, 128), jnp.float32),
    )(h, Wall, bias)


def _tc_h1_body(lo_blk, hi_blk, pa0, pa1, pb0, pb1, out_blk):
    h = pl.program_id(1)
    s = jnp.where(h == 0,
                  lo_blk[...] + pa0[...] + pa1[...],
                  hi_blk[...] + pb0[...] + pb1[...])
    out_blk[...] = jnp.maximum(s, 0.0)


def _tc_h1(y1lo, y1hi, partsA, partsB):
    return pl.pallas_call(
        _tc_h1_body,
        grid=(NNB, 2),
        in_specs=[
            pl.BlockSpec((NBLK, 128), lambda nb, h: (nb, 0)),
            pl.BlockSpec((NBLK, 128), lambda nb, h: (nb, 0)),
            pl.BlockSpec((NBLK, 128), lambda nb, h: (nb, 0)),
            pl.BlockSpec((NBLK, 128), lambda nb, h: (NNB + nb, 0)),
            pl.BlockSpec((NBLK, 128), lambda nb, h: (nb, 0)),
            pl.BlockSpec((NBLK, 128), lambda nb, h: (NNB + nb, 0)),
        ],
        out_specs=pl.BlockSpec((NBLK, 128), lambda nb, h: (nb, h)),
        out_shape=jax.ShapeDtypeStruct((N, HID), jnp.float32),
    )(y1lo, y1hi, partsA, partsA, partsB, partsB)


def _tc_dec_prep_body(root_blk, p0_blk, p1_blk, w_blk, b_blk, out_blk):
    g = pl.program_id(0)
    h2 = root_blk[...] + p0_blk[...] + p1_blk[...]
    y = jnp.dot(h2, w_blk[...], preferred_element_type=jnp.float32)
    out_blk[...] = jnp.where(g == 0, y + b_blk[...], y)


def _tc_dec_prep(y2all, parts, dec_w1, dec_b1):
    return pl.pallas_call(
        _tc_dec_prep_body,
        grid=(2, NNB),
        in_specs=[
            pl.BlockSpec((NBLK, F_OUT), lambda g, nb: (nb, 0)),
            pl.BlockSpec((NBLK, F_OUT), lambda g, nb: (nb, 0)),
            pl.BlockSpec((NBLK, F_OUT), lambda g, nb: (NNB + nb, 0)),
            pl.BlockSpec((F_OUT, F_OUT), lambda g, nb: (g, 0)),
            pl.BlockSpec((1, F_OUT), lambda g, nb: (0, 0)),
        ],
        out_specs=pl.BlockSpec((NBLK, F_OUT), lambda g, nb: (g * NNB + nb, 0)),
        out_shape=jax.ShapeDtypeStruct((2 * N, F_OUT), jnp.float32),
    )(y2all, parts, parts, dec_w1, dec_b1)


ZBLK = 1600


def _tc_out_body(z_blk, w_blk, b_blk, out_blk):
    out_blk[...] = (
        jnp.dot(z_blk[...], w_blk[...], preferred_element_type=jnp.float32)
        + b_blk[...]
    )


def _tc_out(z, dec_w2, dec_b2):
    return pl.pallas_call(
        _tc_out_body,
        grid=(P_PAD // ZBLK,),
        in_specs=[
            pl.BlockSpec((ZBLK, F_OUT), lambda i: (i, 0)),
            pl.BlockSpec((F_OUT, NCLS), lambda i: (0, 0)),
            pl.BlockSpec((1, NCLS), lambda i: (0, 0)),
        ],
        out_specs=pl.BlockSpec((ZBLK, NCLS), lambda i: (i, 0)),
        out_shape=jax.ShapeDtypeStruct((P_PAD, NCLS), jnp.float32),
    )(z, dec_w2, dec_b2)


@jax.jit
def _run(x, edge_index, edge_type, pairs, W1, root1, b1, W2, root2, b2,
         dec_w1, dec_b1, dec_w2, dec_b2):
    ei2 = jnp.asarray(edge_index, jnp.int32).reshape(2, E // 128, 128)
    et2 = jnp.asarray(edge_type, jnp.int32).reshape(E // 128, 128)
    gix2, d2 = _tc_eidx(ei2, et2)
    gix = gix2.reshape(E)
    d32 = d2.reshape(E)
    pairs32 = jnp.asarray(pairs, jnp.int32)
    pairs_p = jnp.concatenate(
        [pairs32, jnp.zeros((P_PAD - P, 2), jnp.int32)], axis=0)
    psrc = pairs_p[:, 0]
    pdst = pairs_p[:, 1]

    root1r = root1.reshape(1, F_IN, HID)
    W1lo = jnp.concatenate([root1r[:, :, :128], W1[:, :, :128]], axis=0)
    W1hi = jnp.concatenate([root1r[:, :, 128:], W1[:, :, 128:]], axis=0)
    y1lo = _tc_transform(x, W1lo, b1[:128].reshape(1, 128))
    y1hi = _tc_transform(x, W1hi, b1[128:].reshape(1, 128))
    partsA = _sc_seg(y1lo, gix, d32)
    y1hi2, _ = lax.optimization_barrier((y1hi, partsA))
    partsB = _sc_seg(y1hi2, gix, d32)
    h1 = _tc_h1(y1lo, y1hi2, partsA, partsB)

    W2all = jnp.concatenate([root2.reshape(1, HID, F_OUT), W2], axis=0)
    y2all = _tc_transform(h1, W2all, b2.reshape(1, F_OUT))
    parts2 = _sc_seg(y2all, gix, d32)

    hAB = _tc_dec_prep(y2all, parts2, dec_w1, dec_b1.reshape(1, F_OUT))
    z = _sc_dec(hAB, psrc, pdst)
    logits = _tc_out(z, dec_w2, dec_b2.reshape(1, NCLS))
    return logits[:P]


def kernel(x, edge_index, edge_type, pairs, W1, root1, b1, W2, root2, b2,
           dec_w1, dec_b1, dec_w2, dec_b2):
    return _run(x, edge_index, edge_type, pairs, W1, root1, b1, W2, root2, b2,
                dec_w1, dec_b1, dec_w2, dec_b2)


# speedup vs baseline: 30.2077x; 1.1898x over previous
import jax
import jax.numpy as jnp
from jax import lax
from jax.experimental import pallas as pl
from jax.experimental.pallas import tpu as pltpu
from jax.experimental.pallas import tpu_sc as plsc

N = 10000
E = 320000
R = 8
F_IN = 128
HID = 256
F_OUT = 128
NCLS = 16
P = 100000
P_PAD = 102400

NC = 2
NS = 16


BE = 80


def _make_sc_seg():
    ept = E // (NC * NS)
    nb = ept // BE
    assert nb % 2 == 1 and nb >= 3

    def body(table, gix, d32, out, gidx0, gidx1, sidx0, sidx1,
             gall, dall, rows0, rows1, zbuf, destbuf, sem0, sem1):
        core = lax.axis_index("c")
        tid = lax.axis_index("s")
        base_e = (core * NS + tid) * ept

        @pl.loop(0, 40)
        def _(i):
            for j in range(F_OUT // 16):
                zbuf[i, pl.ds(j * 16, 16)] = jnp.zeros((16,), jnp.float32)

        zrows = 1000

        @pl.when(tid < 10)
        def _():
            @pl.loop(0, zrows // 40)
            def _(k):
                pltpu.sync_copy(zbuf,
                                destbuf.at[pl.ds(tid * zrows + k * 40, 40)])

        pltpu.sync_copy(gix.at[pl.ds(base_e, ept)], gall)
        pltpu.sync_copy(d32.at[pl.ds(base_e, ept)], dall)
        plsc.subcore_barrier()

        def fill(slotg, slots, b):
            @pl.loop(0, BE // 16)
            def _(i):
                s = pl.ds(i * 16, 16)
                t = pl.ds(b * BE + i * 16, 16)
                slotg[s] = gall[t]
                slots[s] = dall[t]

        fill(gidx0, sidx0, 0)
        pltpu.async_copy(table.at[gidx0], rows0, sem0)

        @pl.loop(0, (nb - 1) // 2)
        def _(p):
            fill(gidx1, sidx1, 2 * p + 1)
            pltpu.async_copy(table.at[gidx1], rows1, sem1)

            pltpu.make_async_copy(table.at[gidx0], rows0, sem0).wait()
            pltpu.sync_copy(rows0, destbuf.at[sidx0], add=True)

            fill(gidx0, sidx0, 2 * p + 2)
            pltpu.async_copy(table.at[gidx0], rows0, sem0)

            pltpu.make_async_copy(table.at[gidx1], rows1, sem1).wait()
            pltpu.sync_copy(rows1, destbuf.at[sidx1], add=True)

        pltpu.make_async_copy(table.at[gidx0], rows0, sem0).wait()
        pltpu.sync_copy(rows0, destbuf.at[sidx0], add=True)

        plsc.subcore_barrier()

        @pl.when(tid < 10)
        def _():
            pltpu.sync_copy(destbuf.at[pl.ds(tid * zrows, zrows)],
                            out.at[pl.ds(core * N + tid * zrows, zrows)])

        plsc.subcore_barrier()

    mesh = plsc.VectorSubcoreMesh(core_axis_name="c", subcore_axis_name="s")

    def run(table, gix, d32):
        return pl.kernel(
            body,
            out_type=jax.ShapeDtypeStruct((NC * N, F_OUT), jnp.float32),
            mesh=mesh,
            scratch_types=[
                pltpu.VMEM((BE,), jnp.int32),
                pltpu.VMEM((BE,), jnp.int32),
                pltpu.VMEM((BE,), jnp.int32),
                pltpu.VMEM((BE,), jnp.int32),
                pltpu.VMEM((E // (NC * NS),), jnp.int32),
                pltpu.VMEM((E // (NC * NS),), jnp.int32),
                pltpu.VMEM((BE, F_OUT), jnp.float32),
                pltpu.VMEM((BE, F_OUT), jnp.float32),
                pltpu.VMEM((40, F_OUT), jnp.float32),
                pltpu.VMEM_SHARED((N, F_OUT), jnp.float32),
                pltpu.SemaphoreType.DMA,
                pltpu.SemaphoreType.DMA,
            ],
        )(table, gix, d32)

    return run


_sc_seg = _make_sc_seg()


PPT = P_PAD // (NC * NS)
B3 = 320
NB3 = PPT // B3


def _sc_dec_body(hAB, psrc, pdst, z, idxa, idxb, tmp, u, v, sem, sem2):
    core = lax.axis_index("c")
    tid = lax.axis_index("s")
    w = core * NS + tid
    base_p = w * PPT

    @pl.loop(0, NB3)
    def _(b):
        off = base_p + b * B3
        pltpu.sync_copy(psrc.at[pl.ds(off, B3)], idxa)
        pltpu.sync_copy(pdst.at[pl.ds(off, B3)], tmp)

        @pl.loop(0, B3 // 16)
        def _(i):
            s = pl.ds(i * 16, 16)
            idxb[s] = tmp[s] + N

        pltpu.async_copy(hAB.at[idxa], u, sem)
        pltpu.async_copy(hAB.at[idxb], v, sem2)
        pltpu.make_async_copy(hAB.at[idxa], u, sem).wait()
        pltpu.make_async_copy(hAB.at[idxb], v, sem2).wait()

        @pl.loop(0, B3, unroll=4)
        def _(i):
            for j in range(F_OUT // 16):
                s = pl.ds(j * 16, 16)
                u[i, s] = jnp.maximum(u[i, s] + v[i, s], 0.0)

        pltpu.sync_copy(u, z.at[pl.ds(off, B3)])


def _sc_dec(hAB, psrc, pdst):
    mesh = plsc.VectorSubcoreMesh(core_axis_name="c", subcore_axis_name="s")
    return pl.kernel(
        _sc_dec_body,
        out_type=jax.ShapeDtypeStruct((P_PAD, F_OUT), jnp.float32),
        mesh=mesh,
        scratch_types=[
            pltpu.VMEM((B3,), jnp.int32),
            pltpu.VMEM((B3,), jnp.int32),
            pltpu.VMEM((B3,), jnp.int32),
            pltpu.VMEM((B3, F_OUT), jnp.float32),
            pltpu.VMEM((B3, F_OUT), jnp.float32),
            pltpu.SemaphoreType.DMA,
            pltpu.SemaphoreType.DMA,
        ],
    )(hAB, psrc, pdst)


NBLK = 1000
NNB = N // NBLK


def _tc_eidx_body(ei_blk, et_blk, gix_blk, d_blk):
    gix_blk[...] = et_blk[...] * N + ei_blk[0] + N
    d_blk[...] = ei_blk[1]


def _tc_eidx(edge_index2, edge_type2):
    eb = 2500
    return pl.pallas_call(
        _tc_eidx_body,
        grid=(E // (eb * 128),),
        in_specs=[
            pl.BlockSpec((2, eb, 128), lambda i: (0, i, 0)),
            pl.BlockSpec((eb, 128), lambda i: (i, 0)),
        ],
        out_specs=[
            pl.BlockSpec((eb, 128), lambda i: (i, 0)),
            pl.BlockSpec((eb, 128), lambda i: (i, 0)),
        ],
        out_shape=[
            jax.ShapeDtypeStruct((E // 128, 128), jnp.int32),
            jax.ShapeDtypeStruct((E // 128, 128), jnp.int32),
        ],
    )(edge_index2, edge_type2)


def _tc_transform_body(h_blk, w_blk, b_blk, out_blk):
    r = pl.program_id(1)
    y = jnp.dot(h_blk[...], w_blk[0], preferred_element_type=jnp.float32)
    out_blk[...] = jnp.where(r == 0, y + b_blk[...], y)


def _tc_transform(h, Wall, bias):
    in_dim = h.shape[1]
    return pl.pallas_call(
        _tc_transform_body,
        grid=(NNB, R + 1),
        in_specs=[
            pl.BlockSpec((NBLK, in_dim), lambda nb, r: (nb, 0)),
            pl.BlockSpec((1, in_dim, 128), lambda nb, r: (r, 0, 0)),
            pl.BlockSpec((1, 128), lambda nb, r: (0, 0)),
        ],
        out_specs=pl.BlockSpec((NBLK, 128), lambda nb, r: (r * NNB + nb, 0)),
        out_shape=jax.ShapeDtypeStruct(((R + 1) * N, 128), jnp.float32),
    )(h, Wall, bias)


def _tc_h1_body(lo_blk, hi_blk, pa0, pa1, pb0, pb1, out_blk):
    h = pl.program_id(1)
    s = jnp.where(h == 0,
                  lo_blk[...] + pa0[...] + pa1[...],
                  hi_blk[...] + pb0[...] + pb1[...])
    out_blk[...] = jnp.maximum(s, 0.0)


def _tc_h1(y1lo, y1hi, partsA, partsB):
    return pl.pallas_call(
        _tc_h1_body,
        grid=(NNB, 2),
        in_specs=[
            pl.BlockSpec((NBLK, 128), lambda nb, h: (nb, 0)),
            pl.BlockSpec((NBLK, 128), lambda nb, h: (nb, 0)),
            pl.BlockSpec((NBLK, 128), lambda nb, h: (nb, 0)),
            pl.BlockSpec((NBLK, 128), lambda nb, h: (NNB + nb, 0)),
            pl.BlockSpec((NBLK, 128), lambda nb, h: (nb, 0)),
            pl.BlockSpec((NBLK, 128), lambda nb, h: (NNB + nb, 0)),
        ],
        out_specs=pl.BlockSpec((NBLK, 128), lambda nb, h: (nb, h)),
        out_shape=jax.ShapeDtypeStruct((N, HID), jnp.float32),
    )(y1lo, y1hi, partsA, partsA, partsB, partsB)


def _tc_dec_prep_body(root_blk, p0_blk, p1_blk, w_blk, b_blk, out_blk):
    g = pl.program_id(0)
    h2 = root_blk[...] + p0_blk[...] + p1_blk[...]
    y = jnp.dot(h2, w_blk[...], preferred_element_type=jnp.float32)
    out_blk[...] = jnp.where(g == 0, y + b_blk[...], y)


def _tc_dec_prep(y2all, parts, dec_w1, dec_b1):
    return pl.pallas_call(
        _tc_dec_prep_body,
        grid=(2, NNB),
        in_specs=[
            pl.BlockSpec((NBLK, F_OUT), lambda g, nb: (nb, 0)),
            pl.BlockSpec((NBLK, F_OUT), lambda g, nb: (nb, 0)),
            pl.BlockSpec((NBLK, F_OUT), lambda g, nb: (NNB + nb, 0)),
            pl.BlockSpec((F_OUT, F_OUT), lambda g, nb: (g, 0)),
            pl.BlockSpec((1, F_OUT), lambda g, nb: (0, 0)),
        ],
        out_specs=pl.BlockSpec((NBLK, F_OUT), lambda g, nb: (g * NNB + nb, 0)),
        out_shape=jax.ShapeDtypeStruct((2 * N, F_OUT), jnp.float32),
    )(y2all, parts, parts, dec_w1, dec_b1)


ZBLK = 1600


def _tc_out_body(z_blk, w_blk, b_blk, out_blk):
    out_blk[...] = (
        jnp.dot(z_blk[...], w_blk[...], preferred_element_type=jnp.float32)
        + b_blk[...]
    )


def _tc_out(z, dec_w2, dec_b2):
    return pl.pallas_call(
        _tc_out_body,
        grid=(P_PAD // ZBLK,),
        in_specs=[
            pl.BlockSpec((ZBLK, F_OUT), lambda i: (i, 0)),
            pl.BlockSpec((F_OUT, NCLS), lambda i: (0, 0)),
            pl.BlockSpec((1, NCLS), lambda i: (0, 0)),
        ],
        out_specs=pl.BlockSpec((ZBLK, NCLS), lambda i: (i, 0)),
        out_shape=jax.ShapeDtypeStruct((P_PAD, NCLS), jnp.float32),
    )(z, dec_w2, dec_b2)


@jax.jit
def _run(x, edge_index, edge_type, pairs, W1, root1, b1, W2, root2, b2,
         dec_w1, dec_b1, dec_w2, dec_b2):
    ei2 = jnp.asarray(edge_index, jnp.int32).reshape(2, E // 128, 128)
    et2 = jnp.asarray(edge_type, jnp.int32).reshape(E // 128, 128)
    gix2, d2 = _tc_eidx(ei2, et2)
    gix = gix2.reshape(E)
    d32 = d2.reshape(E)
    pairs32 = jnp.asarray(pairs, jnp.int32)
    pairs_p = jnp.concatenate(
        [pairs32, jnp.zeros((P_PAD - P, 2), jnp.int32)], axis=0)
    psrc = pairs_p[:, 0]
    pdst = pairs_p[:, 1]

    root1r = root1.reshape(1, F_IN, HID)
    W1lo = jnp.concatenate([root1r[:, :, :128], W1[:, :, :128]], axis=0)
    W1hi = jnp.concatenate([root1r[:, :, 128:], W1[:, :, 128:]], axis=0)
    y1lo = _tc_transform(x, W1lo, b1[:128].reshape(1, 128))
    y1hi = _tc_transform(x, W1hi, b1[128:].reshape(1, 128))
    partsA = _sc_seg(y1lo, gix, d32)
    y1hi2, _ = lax.optimization_barrier((y1hi, partsA))
    partsB = _sc_seg(y1hi2, gix, d32)
    h1 = _tc_h1(y1lo, y1hi2, partsA, partsB)

    W2all = jnp.concatenate([root2.reshape(1, HID, F_OUT), W2], axis=0)
    y2all = _tc_transform(h1, W2all, b2.reshape(1, F_OUT))
    parts2 = _sc_seg(y2all, gix, d32)

    hAB = _tc_dec_prep(y2all, parts2, dec_w1, dec_b1.reshape(1, F_OUT))
    z = _sc_dec(hAB, psrc, pdst)
    logits = _tc_out(z, dec_w2, dec_b2.reshape(1, NCLS))
    return logits[:P]


def kernel(x, edge_index, edge_type, pairs, W1, root1, b1, W2, root2, b2,
           dec_w1, dec_b1, dec_w2, dec_b2):
    return _run(x, edge_index, edge_type, pairs, W1, root1, b1, W2, root2, b2,
                dec_w1, dec_b1, dec_w2, dec_b2)


# speedup vs baseline: 33.9384x; 1.1235x over previous
import jax
import jax.numpy as jnp
from jax import lax
from jax.experimental import pallas as pl
from jax.experimental.pallas import tpu as pltpu
from jax.experimental.pallas import tpu_sc as plsc

N = 10000
E = 320000
R = 8
F_IN = 128
HID = 256
F_OUT = 128
NCLS = 16
P = 100000
P_PAD = 102400

NC = 2
NS = 16


BE = 80


def _make_sc_seg():
    ept = E // (NC * NS)
    nb = ept // BE
    assert nb % 2 == 1 and nb >= 3

    def body(table, gix, d32, out, gidx0, gidx1, sidx0, sidx1,
             gall, dall, rows0, rows1, zbuf, destbuf, sem0, sem1):
        core = lax.axis_index("c")
        tid = lax.axis_index("s")
        base_e = (core * NS + tid) * ept

        @pl.loop(0, 40)
        def _(i):
            for j in range(F_OUT // 16):
                zbuf[i, pl.ds(j * 16, 16)] = jnp.zeros((16,), jnp.float32)

        zrows = 1000

        @pl.when(tid < 10)
        def _():
            @pl.loop(0, zrows // 40)
            def _(k):
                pltpu.sync_copy(zbuf,
                                destbuf.at[pl.ds(tid * zrows + k * 40, 40)])

        pltpu.sync_copy(gix.at[pl.ds(base_e, ept)], gall)
        pltpu.sync_copy(d32.at[pl.ds(base_e, ept)], dall)
        plsc.subcore_barrier()

        def fill(slotg, slots, b):
            @pl.loop(0, BE // 16)
            def _(i):
                s = pl.ds(i * 16, 16)
                t = pl.ds(b * BE + i * 16, 16)
                slotg[s] = gall[t]
                slots[s] = dall[t]

        fill(gidx0, sidx0, 0)
        pltpu.async_copy(table.at[gidx0], rows0, sem0)

        @pl.loop(0, (nb - 1) // 2)
        def _(p):
            fill(gidx1, sidx1, 2 * p + 1)
            pltpu.async_copy(table.at[gidx1], rows1, sem1)

            pltpu.make_async_copy(table.at[gidx0], rows0, sem0).wait()
            pltpu.sync_copy(rows0, destbuf.at[sidx0], add=True)

            fill(gidx0, sidx0, 2 * p + 2)
            pltpu.async_copy(table.at[gidx0], rows0, sem0)

            pltpu.make_async_copy(table.at[gidx1], rows1, sem1).wait()
            pltpu.sync_copy(rows1, destbuf.at[sidx1], add=True)

        pltpu.make_async_copy(table.at[gidx0], rows0, sem0).wait()
        pltpu.sync_copy(rows0, destbuf.at[sidx0], add=True)

        plsc.subcore_barrier()

        @pl.when(tid < 10)
        def _():
            pltpu.sync_copy(destbuf.at[pl.ds(tid * zrows, zrows)],
                            out.at[pl.ds(core * N + tid * zrows, zrows)])

        plsc.subcore_barrier()

    mesh = plsc.VectorSubcoreMesh(core_axis_name="c", subcore_axis_name="s")

    def run(table, gix, d32):
        return pl.kernel(
            body,
            out_type=jax.ShapeDtypeStruct((NC * N, F_OUT), jnp.float32),
            mesh=mesh,
            scratch_types=[
                pltpu.VMEM((BE,), jnp.int32),
                pltpu.VMEM((BE,), jnp.int32),
                pltpu.VMEM((BE,), jnp.int32),
                pltpu.VMEM((BE,), jnp.int32),
                pltpu.VMEM((E // (NC * NS),), jnp.int32),
                pltpu.VMEM((E // (NC * NS),), jnp.int32),
                pltpu.VMEM((BE, F_OUT), jnp.float32),
                pltpu.VMEM((BE, F_OUT), jnp.float32),
                pltpu.VMEM((40, F_OUT), jnp.float32),
                pltpu.VMEM_SHARED((N, F_OUT), jnp.float32),
                pltpu.SemaphoreType.DMA,
                pltpu.SemaphoreType.DMA,
            ],
        )(table, gix, d32)

    return run


_sc_seg = _make_sc_seg()


PPT = P_PAD // (NC * NS)
B3 = 160
NB3 = PPT // B3


def _sc_dec_body(hAB, psrc, pdst, z, pall, qall, ia0, ib0, ia1, ib1,
                 u0, v0, u1, v1, sa0, sb0, sa1, sb1):
    core = lax.axis_index("c")
    tid = lax.axis_index("s")
    w = core * NS + tid
    base_p = w * PPT

    pltpu.sync_copy(psrc.at[pl.ds(base_p, PPT)], pall)
    pltpu.sync_copy(pdst.at[pl.ds(base_p, PPT)], qall)

    def fill(ia, ib, b):
        @pl.loop(0, B3 // 16)
        def _(i):
            s = pl.ds(i * 16, 16)
            t = pl.ds(b * B3 + i * 16, 16)
            ia[s] = pall[t]
            ib[s] = qall[t] + N

    def fire(ia, ib, u, v, sa, sb):
        pltpu.async_copy(hAB.at[ia], u, sa)
        pltpu.async_copy(hAB.at[ib], v, sb)

    def finish(ia, ib, u, v, sa, sb, b):
        pltpu.make_async_copy(hAB.at[ia], u, sa).wait()
        pltpu.make_async_copy(hAB.at[ib], v, sb).wait()

        @pl.loop(0, B3, unroll=4)
        def _(i):
            for j in range(F_OUT // 16):
                s = pl.ds(j * 16, 16)
                u[i, s] = jnp.maximum(u[i, s] + v[i, s], 0.0)

        pltpu.sync_copy(u, z.at[pl.ds(base_p + b * B3, B3)])

    fill(ia0, ib0, 0)
    fire(ia0, ib0, u0, v0, sa0, sb0)

    @pl.loop(0, NB3 // 2)
    def _(p):
        fill(ia1, ib1, 2 * p + 1)
        fire(ia1, ib1, u1, v1, sa1, sb1)
        finish(ia0, ib0, u0, v0, sa0, sb0, 2 * p)

        @pl.when(p < NB3 // 2 - 1)
        def _():
            fill(ia0, ib0, 2 * p + 2)
            fire(ia0, ib0, u0, v0, sa0, sb0)

        finish(ia1, ib1, u1, v1, sa1, sb1, 2 * p + 1)


def _sc_dec(hAB, psrc, pdst):
    mesh = plsc.VectorSubcoreMesh(core_axis_name="c", subcore_axis_name="s")
    return pl.kernel(
        _sc_dec_body,
        out_type=jax.ShapeDtypeStruct((P_PAD, F_OUT), jnp.float32),
        mesh=mesh,
        scratch_types=[
            pltpu.VMEM((PPT,), jnp.int32),
            pltpu.VMEM((PPT,), jnp.int32),
            pltpu.VMEM((B3,), jnp.int32),
            pltpu.VMEM((B3,), jnp.int32),
            pltpu.VMEM((B3,), jnp.int32),
            pltpu.VMEM((B3,), jnp.int32),
            pltpu.VMEM((B3, F_OUT), jnp.float32),
            pltpu.VMEM((B3, F_OUT), jnp.float32),
            pltpu.VMEM((B3, F_OUT), jnp.float32),
            pltpu.VMEM((B3, F_OUT), jnp.float32),
            pltpu.SemaphoreType.DMA,
            pltpu.SemaphoreType.DMA,
            pltpu.SemaphoreType.DMA,
            pltpu.SemaphoreType.DMA,
        ],
    )(hAB, psrc, pdst)


NBLK = 1000
NNB = N // NBLK


def _tc_eidx_body(ei_blk, et_blk, gix_blk, d_blk):
    gix_blk[...] = et_blk[...] * N + ei_blk[0] + N
    d_blk[...] = ei_blk[1]


def _tc_eidx(edge_index2, edge_type2):
    eb = 2500
    return pl.pallas_call(
        _tc_eidx_body,
        grid=(E // (eb * 128),),
        in_specs=[
            pl.BlockSpec((2, eb, 128), lambda i: (0, i, 0)),
            pl.BlockSpec((eb, 128), lambda i: (i, 0)),
        ],
        out_specs=[
            pl.BlockSpec((eb, 128), lambda i: (i, 0)),
            pl.BlockSpec((eb, 128), lambda i: (i, 0)),
        ],
        out_shape=[
            jax.ShapeDtypeStruct((E // 128, 128), jnp.int32),
            jax.ShapeDtypeStruct((E // 128, 128), jnp.int32),
        ],
    )(edge_index2, edge_type2)


def _tc_transform_body(h_blk, w_blk, b_blk, out_blk):
    r = pl.program_id(1)
    y = jnp.dot(h_blk[...], w_blk[0], preferred_element_type=jnp.float32)
    out_blk[...] = jnp.where(r == 0, y + b_blk[...], y)


def _tc_transform(h, Wall, bias):
    in_dim = h.shape[1]
    return pl.pallas_call(
        _tc_transform_body,
        grid=(NNB, R + 1),
        in_specs=[
            pl.BlockSpec((NBLK, in_dim), lambda nb, r: (nb, 0)),
            pl.BlockSpec((1, in_dim, 128), lambda nb, r: (r, 0, 0)),
            pl.BlockSpec((1, 128), lambda nb, r: (0, 0)),
        ],
        out_specs=pl.BlockSpec((NBLK, 128), lambda nb, r: (r * NNB + nb, 0)),
        out_shape=jax.ShapeDtypeStruct(((R + 1) * N, 128), jnp.float32),
    )(h, Wall, bias)


def _tc_h1_body(lo_blk, hi_blk, pa0, pa1, pb0, pb1, out_blk):
    h = pl.program_id(1)
    s = jnp.where(h == 0,
                  lo_blk[...] + pa0[...] + pa1[...],
                  hi_blk[...] + pb0[...] + pb1[...])
    out_blk[...] = jnp.maximum(s, 0.0)


def _tc_h1(y1lo, y1hi, partsA, partsB):
    return pl.pallas_call(
        _tc_h1_body,
        grid=(NNB, 2),
        in_specs=[
            pl.BlockSpec((NBLK, 128), lambda nb, h: (nb, 0)),
            pl.BlockSpec((NBLK, 128), lambda nb, h: (nb, 0)),
            pl.BlockSpec((NBLK, 128), lambda nb, h: (nb, 0)),
            pl.BlockSpec((NBLK, 128), lambda nb, h: (NNB + nb, 0)),
            pl.BlockSpec((NBLK, 128), lambda nb, h: (nb, 0)),
            pl.BlockSpec((NBLK, 128), lambda nb, h: (NNB + nb, 0)),
        ],
        out_specs=pl.BlockSpec((NBLK, 128), lambda nb, h: (nb, h)),
        out_shape=jax.ShapeDtypeStruct((N, HID), jnp.float32),
    )(y1lo, y1hi, partsA, partsA, partsB, partsB)


def _tc_dec_prep_body(root_blk, p0_blk, p1_blk, w_blk, b_blk, out_blk):
    g = pl.program_id(0)
    h2 = root_blk[...] + p0_blk[...] + p1_blk[...]
    y = jnp.dot(h2, w_blk[...], preferred_element_type=jnp.float32)
    out_blk[...] = jnp.where(g == 0, y + b_blk[...], y)


def _tc_dec_prep(y2all, parts, dec_w1, dec_b1):
    return pl.pallas_call(
        _tc_dec_prep_body,
        grid=(2, NNB),
        in_specs=[
            pl.BlockSpec((NBLK, F_OUT), lambda g, nb: (nb, 0)),
            pl.BlockSpec((NBLK, F_OUT), lambda g, nb: (nb, 0)),
            pl.BlockSpec((NBLK, F_OUT), lambda g, nb: (NNB + nb, 0)),
            pl.BlockSpec((F_OUT, F_OUT), lambda g, nb: (g, 0)),
            pl.BlockSpec((1, F_OUT), lambda g, nb: (0, 0)),
        ],
        out_specs=pl.BlockSpec((NBLK, F_OUT), lambda g, nb: (g * NNB + nb, 0)),
        out_shape=jax.ShapeDtypeStruct((2 * N, F_OUT), jnp.float32),
    )(y2all, parts, parts, dec_w1, dec_b1)


ZBLK = 1600


def _tc_out_body(z_blk, w_blk, b_blk, out_blk):
    out_blk[...] = (
        jnp.dot(z_blk[...], w_blk[...], preferred_element_type=jnp.float32)
        + b_blk[...]
    )


def _tc_out(z, dec_w2, dec_b2):
    return pl.pallas_call(
        _tc_out_body,
        grid=(P_PAD // ZBLK,),
        in_specs=[
            pl.BlockSpec((ZBLK, F_OUT), lambda i: (i, 0)),
            pl.BlockSpec((F_OUT, NCLS), lambda i: (0, 0)),
            pl.BlockSpec((1, NCLS), lambda i: (0, 0)),
        ],
        out_specs=pl.BlockSpec((ZBLK, NCLS), lambda i: (i, 0)),
        out_shape=jax.ShapeDtypeStruct((P_PAD, NCLS), jnp.float32),
    )(z, dec_w2, dec_b2)


@jax.jit
def _run(x, edge_index, edge_type, pairs, W1, root1, b1, W2, root2, b2,
         dec_w1, dec_b1, dec_w2, dec_b2):
    ei2 = jnp.asarray(edge_index, jnp.int32).reshape(2, E // 128, 128)
    et2 = jnp.asarray(edge_type, jnp.int32).reshape(E // 128, 128)
    gix2, d2 = _tc_eidx(ei2, et2)
    gix = gix2.reshape(E)
    d32 = d2.reshape(E)
    pairs32 = jnp.asarray(pairs, jnp.int32)
    pairs_p = jnp.concatenate(
        [pairs32, jnp.zeros((P_PAD - P, 2), jnp.int32)], axis=0)
    psrc = pairs_p[:, 0]
    pdst = pairs_p[:, 1]

    root1r = root1.reshape(1, F_IN, HID)
    W1lo = jnp.concatenate([root1r[:, :, :128], W1[:, :, :128]], axis=0)
    W1hi = jnp.concatenate([root1r[:, :, 128:], W1[:, :, 128:]], axis=0)
    y1lo = _tc_transform(x, W1lo, b1[:128].reshape(1, 128))
    y1hi = _tc_transform(x, W1hi, b1[128:].reshape(1, 128))
    partsA = _sc_seg(y1lo, gix, d32)
    y1hi2, _ = lax.optimization_barrier((y1hi, partsA))
    partsB = _sc_seg(y1hi2, gix, d32)
    h1 = _tc_h1(y1lo, y1hi2, partsA, partsB)

    W2all = jnp.concatenate([root2.reshape(1, HID, F_OUT), W2], axis=0)
    y2all = _tc_transform(h1, W2all, b2.reshape(1, F_OUT))
    parts2 = _sc_seg(y2all, gix, d32)

    hAB = _tc_dec_prep(y2all, parts2, dec_w1, dec_b1.reshape(1, F_OUT))
    z = _sc_dec(hAB, psrc, pdst)
    logits = _tc_out(z, dec_w2, dec_b2.reshape(1, NCLS))
    return logits[:P]


def kernel(x, edge_index, edge_type, pairs, W1, root1, b1, W2, root2, b2,
           dec_w1, dec_b1, dec_w2, dec_b2):
    return _run(x, edge_index, edge_type, pairs, W1, root1, b1, W2, root2, b2,
                dec_w1, dec_b1, dec_w2, dec_b2)


# speedup vs baseline: 34.1975x; 1.0076x over previous
import jax
import jax.numpy as jnp
from jax import lax
from jax.experimental import pallas as pl
from jax.experimental.pallas import tpu as pltpu
from jax.experimental.pallas import tpu_sc as plsc

N = 10000
E = 320000
R = 8
F_IN = 128
HID = 256
F_OUT = 128
NCLS = 16
P = 100000
P_PAD = 102400

NC = 2
NS = 16


BE = 80


def _make_sc_seg():
    ept = E // (NC * NS)
    nb = ept // BE
    assert nb % 2 == 1 and nb >= 3

    def body(table, gix, d32, out, gidx0, gidx1, sidx0, sidx1,
             gall, dall, rows0, rows1, zbuf, destbuf, sem0, sem1):
        core = lax.axis_index("c")
        tid = lax.axis_index("s")
        base_e = (core * NS + tid) * ept

        @pl.loop(0, 40)
        def _(i):
            for j in range(F_OUT // 16):
                zbuf[i, pl.ds(j * 16, 16)] = jnp.zeros((16,), jnp.float32)

        zrows = 640

        @pl.loop(0, zrows // 40)
        def _(k):
            pltpu.sync_copy(zbuf,
                            destbuf.at[pl.ds(tid * zrows + k * 40, 40)])

        pltpu.sync_copy(gix.at[pl.ds(base_e, ept)], gall)
        pltpu.sync_copy(d32.at[pl.ds(base_e, ept)], dall)
        plsc.subcore_barrier()

        def fill(slotg, slots, b):
            @pl.loop(0, BE // 16)
            def _(i):
                s = pl.ds(i * 16, 16)
                t = pl.ds(b * BE + i * 16, 16)
                slotg[s] = gall[t]
                slots[s] = dall[t]

        fill(gidx0, sidx0, 0)
        pltpu.async_copy(table.at[gidx0], rows0, sem0)

        @pl.loop(0, (nb - 1) // 2)
        def _(p):
            fill(gidx1, sidx1, 2 * p + 1)
            pltpu.async_copy(table.at[gidx1], rows1, sem1)

            pltpu.make_async_copy(table.at[gidx0], rows0, sem0).wait()
            pltpu.sync_copy(rows0, destbuf.at[sidx0], add=True)

            fill(gidx0, sidx0, 2 * p + 2)
            pltpu.async_copy(table.at[gidx0], rows0, sem0)

            pltpu.make_async_copy(table.at[gidx1], rows1, sem1).wait()
            pltpu.sync_copy(rows1, destbuf.at[sidx1], add=True)

        pltpu.make_async_copy(table.at[gidx0], rows0, sem0).wait()
        pltpu.sync_copy(rows0, destbuf.at[sidx0], add=True)

        plsc.subcore_barrier()

        @pl.when(tid < 15)
        def _():
            pltpu.sync_copy(destbuf.at[pl.ds(tid * zrows, zrows)],
                            out.at[pl.ds(core * N + tid * zrows, zrows)])

        @pl.when(tid == 15)
        def _():
            pltpu.sync_copy(destbuf.at[pl.ds(15 * zrows, 400)],
                            out.at[pl.ds(core * N + 15 * zrows, 400)])

        plsc.subcore_barrier()

    mesh = plsc.VectorSubcoreMesh(core_axis_name="c", subcore_axis_name="s")

    def run(table, gix, d32):
        return pl.kernel(
            body,
            out_type=jax.ShapeDtypeStruct((NC * N, F_OUT), jnp.float32),
            mesh=mesh,
            scratch_types=[
                pltpu.VMEM((BE,), jnp.int32),
                pltpu.VMEM((BE,), jnp.int32),
                pltpu.VMEM((BE,), jnp.int32),
                pltpu.VMEM((BE,), jnp.int32),
                pltpu.VMEM((E // (NC * NS),), jnp.int32),
                pltpu.VMEM((E // (NC * NS),), jnp.int32),
                pltpu.VMEM((BE, F_OUT), jnp.float32),
                pltpu.VMEM((BE, F_OUT), jnp.float32),
                pltpu.VMEM((40, F_OUT), jnp.float32),
                pltpu.VMEM_SHARED((16 * 640, F_OUT), jnp.float32),
                pltpu.SemaphoreType.DMA,
                pltpu.SemaphoreType.DMA,
            ],
        )(table, gix, d32)

    return run


_sc_seg = _make_sc_seg()


PPT = P_PAD // (NC * NS)
B3 = 160
NB3 = PPT // B3


def _sc_dec_body(hAB, psrc, pdst, z, pall, qall, ia0, ib0, ia1, ib1,
                 u0, v0, u1, v1, sa0, sb0, sa1, sb1):
    core = lax.axis_index("c")
    tid = lax.axis_index("s")
    w = core * NS + tid
    base_p = w * PPT

    pltpu.sync_copy(psrc.at[pl.ds(base_p, PPT)], pall)
    pltpu.sync_copy(pdst.at[pl.ds(base_p, PPT)], qall)

    def fill(ia, ib, b):
        @pl.loop(0, B3 // 16)
        def _(i):
            s = pl.ds(i * 16, 16)
            t = pl.ds(b * B3 + i * 16, 16)
            ia[s] = pall[t]
            ib[s] = qall[t] + N

    def fire(ia, ib, u, v, sa, sb):
        pltpu.async_copy(hAB.at[ia], u, sa)
        pltpu.async_copy(hAB.at[ib], v, sb)

    def finish(ia, ib, u, v, sa, sb, b):
        pltpu.make_async_copy(hAB.at[ia], u, sa).wait()
        pltpu.make_async_copy(hAB.at[ib], v, sb).wait()

        @pl.loop(0, B3, unroll=4)
        def _(i):
            for j in range(F_OUT // 16):
                s = pl.ds(j * 16, 16)
                u[i, s] = jnp.maximum(u[i, s] + v[i, s], 0.0)

        pltpu.sync_copy(u, z.at[pl.ds(base_p + b * B3, B3)])

    fill(ia0, ib0, 0)
    fire(ia0, ib0, u0, v0, sa0, sb0)

    @pl.loop(0, NB3 // 2)
    def _(p):
        fill(ia1, ib1, 2 * p + 1)
        fire(ia1, ib1, u1, v1, sa1, sb1)
        finish(ia0, ib0, u0, v0, sa0, sb0, 2 * p)

        @pl.when(p < NB3 // 2 - 1)
        def _():
            fill(ia0, ib0, 2 * p + 2)
            fire(ia0, ib0, u0, v0, sa0, sb0)

        finish(ia1, ib1, u1, v1, sa1, sb1, 2 * p + 1)


def _sc_dec(hAB, psrc, pdst):
    mesh = plsc.VectorSubcoreMesh(core_axis_name="c", subcore_axis_name="s")
    return pl.kernel(
        _sc_dec_body,
        out_type=jax.ShapeDtypeStruct((P_PAD, F_OUT), jnp.float32),
        mesh=mesh,
        scratch_types=[
            pltpu.VMEM((PPT,), jnp.int32),
            pltpu.VMEM((PPT,), jnp.int32),
            pltpu.VMEM((B3,), jnp.int32),
            pltpu.VMEM((B3,), jnp.int32),
            pltpu.VMEM((B3,), jnp.int32),
            pltpu.VMEM((B3,), jnp.int32),
            pltpu.VMEM((B3, F_OUT), jnp.float32),
            pltpu.VMEM((B3, F_OUT), jnp.float32),
            pltpu.VMEM((B3, F_OUT), jnp.float32),
            pltpu.VMEM((B3, F_OUT), jnp.float32),
            pltpu.SemaphoreType.DMA,
            pltpu.SemaphoreType.DMA,
            pltpu.SemaphoreType.DMA,
            pltpu.SemaphoreType.DMA,
        ],
    )(hAB, psrc, pdst)


NBLK = 1000
NNB = N // NBLK


def _tc_eidx_body(ei_blk, et_blk, gix_blk, d_blk):
    gix_blk[...] = et_blk[...] * N + ei_blk[0] + N
    d_blk[...] = ei_blk[1]


def _tc_eidx(edge_index2, edge_type2):
    eb = 2500
    return pl.pallas_call(
        _tc_eidx_body,
        grid=(E // (eb * 128),),
        in_specs=[
            pl.BlockSpec((2, eb, 128), lambda i: (0, i, 0)),
            pl.BlockSpec((eb, 128), lambda i: (i, 0)),
        ],
        out_specs=[
            pl.BlockSpec((eb, 128), lambda i: (i, 0)),
            pl.BlockSpec((eb, 128), lambda i: (i, 0)),
        ],
        out_shape=[
            jax.ShapeDtypeStruct((E // 128, 128), jnp.int32),
            jax.ShapeDtypeStruct((E // 128, 128), jnp.int32),
        ],
    )(edge_index2, edge_type2)


def _tc_transform_body(h_blk, w_blk, b_blk, out_blk):
    r = pl.program_id(1)
    y = jnp.dot(h_blk[...], w_blk[0], preferred_element_type=jnp.float32)
    out_blk[...] = jnp.where(r == 0, y + b_blk[...], y)


def _tc_transform(h, Wall, bias):
    in_dim = h.shape[1]
    return pl.pallas_call(
        _tc_transform_body,
        grid=(NNB, R + 1),
        in_specs=[
            pl.BlockSpec((NBLK, in_dim), lambda nb, r: (nb, 0)),
            pl.BlockSpec((1, in_dim, 128), lambda nb, r: (r, 0, 0)),
            pl.BlockSpec((1, 128), lambda nb, r: (0, 0)),
        ],
        out_specs=pl.BlockSpec((NBLK, 128), lambda nb, r: (r * NNB + nb, 0)),
        out_shape=jax.ShapeDtypeStruct(((R + 1) * N, 128), jnp.float32),
    )(h, Wall, bias)


def _tc_h1_body(lo_blk, hi_blk, pa0, pa1, pb0, pb1, out_blk):
    h = pl.program_id(1)
    s = jnp.where(h == 0,
                  lo_blk[...] + pa0[...] + pa1[...],
                  hi_blk[...] + pb0[...] + pb1[...])
    out_blk[...] = jnp.maximum(s, 0.0)


def _tc_h1(y1lo, y1hi, partsA, partsB):
    return pl.pallas_call(
        _tc_h1_body,
        grid=(NNB, 2),
        in_specs=[
            pl.BlockSpec((NBLK, 128), lambda nb, h: (nb, 0)),
            pl.BlockSpec((NBLK, 128), lambda nb, h: (nb, 0)),
            pl.BlockSpec((NBLK, 128), lambda nb, h: (nb, 0)),
            pl.BlockSpec((NBLK, 128), lambda nb, h: (NNB + nb, 0)),
            pl.BlockSpec((NBLK, 128), lambda nb, h: (nb, 0)),
            pl.BlockSpec((NBLK, 128), lambda nb, h: (NNB + nb, 0)),
        ],
        out_specs=pl.BlockSpec((NBLK, 128), lambda nb, h: (nb, h)),
        out_shape=jax.ShapeDtypeStruct((N, HID), jnp.float32),
    )(y1lo, y1hi, partsA, partsA, partsB, partsB)


def _tc_dec_prep_body(root_blk, p0_blk, p1_blk, w_blk, b_blk, out_blk):
    g = pl.program_id(0)
    h2 = root_blk[...] + p0_blk[...] + p1_blk[...]
    y = jnp.dot(h2, w_blk[...], preferred_element_type=jnp.float32)
    out_blk[...] = jnp.where(g == 0, y + b_blk[...], y)


def _tc_dec_prep(y2all, parts, dec_w1, dec_b1):
    return pl.pallas_call(
        _tc_dec_prep_body,
        grid=(2, NNB),
        in_specs=[
            pl.BlockSpec((NBLK, F_OUT), lambda g, nb: (nb, 0)),
            pl.BlockSpec((NBLK, F_OUT), lambda g, nb: (nb, 0)),
            pl.BlockSpec((NBLK, F_OUT), lambda g, nb: (NNB + nb, 0)),
            pl.BlockSpec((F_OUT, F_OUT), lambda g, nb: (g, 0)),
            pl.BlockSpec((1, F_OUT), lambda g, nb: (0, 0)),
        ],
        out_specs=pl.BlockSpec((NBLK, F_OUT), lambda g, nb: (g * NNB + nb, 0)),
        out_shape=jax.ShapeDtypeStruct((2 * N, F_OUT), jnp.float32),
    )(y2all, parts, parts, dec_w1, dec_b1)


ZBLK = 1600


def _tc_out_body(z_blk, w_blk, b_blk, out_blk):
    out_blk[...] = (
        jnp.dot(z_blk[...], w_blk[...], preferred_element_type=jnp.float32)
        + b_blk[...]
    )


def _tc_out(z, dec_w2, dec_b2):
    return pl.pallas_call(
        _tc_out_body,
        grid=(P_PAD // ZBLK,),
        in_specs=[
            pl.BlockSpec((ZBLK, F_OUT), lambda i: (i, 0)),
            pl.BlockSpec((F_OUT, NCLS), lambda i: (0, 0)),
            pl.BlockSpec((1, NCLS), lambda i: (0, 0)),
        ],
        out_specs=pl.BlockSpec((ZBLK, NCLS), lambda i: (i, 0)),
        out_shape=jax.ShapeDtypeStruct((P_PAD, NCLS), jnp.float32),
    )(z, dec_w2, dec_b2)


@jax.jit
def _run(x, edge_index, edge_type, pairs, W1, root1, b1, W2, root2, b2,
         dec_w1, dec_b1, dec_w2, dec_b2):
    ei2 = jnp.asarray(edge_index, jnp.int32).reshape(2, E // 128, 128)
    et2 = jnp.asarray(edge_type, jnp.int32).reshape(E // 128, 128)
    gix2, d2 = _tc_eidx(ei2, et2)
    gix = gix2.reshape(E)
    d32 = d2.reshape(E)
    pairs32 = jnp.asarray(pairs, jnp.int32)
    pairs_p = jnp.concatenate(
        [pairs32, jnp.zeros((P_PAD - P, 2), jnp.int32)], axis=0)
    psrc = pairs_p[:, 0]
    pdst = pairs_p[:, 1]

    root1r = root1.reshape(1, F_IN, HID)
    W1lo = jnp.concatenate([root1r[:, :, :128], W1[:, :, :128]], axis=0)
    W1hi = jnp.concatenate([root1r[:, :, 128:], W1[:, :, 128:]], axis=0)
    y1lo = _tc_transform(x, W1lo, b1[:128].reshape(1, 128))
    y1hi = _tc_transform(x, W1hi, b1[128:].reshape(1, 128))
    partsA = _sc_seg(y1lo, gix, d32)
    y1hi2, _ = lax.optimization_barrier((y1hi, partsA))
    partsB = _sc_seg(y1hi2, gix, d32)
    h1 = _tc_h1(y1lo, y1hi2, partsA, partsB)

    W2all = jnp.concatenate([root2.reshape(1, HID, F_OUT), W2], axis=0)
    y2all = _tc_transform(h1, W2all, b2.reshape(1, F_OUT))
    parts2 = _sc_seg(y2all, gix, d32)

    hAB = _tc_dec_prep(y2all, parts2, dec_w1, dec_b1.reshape(1, F_OUT))
    z = _sc_dec(hAB, psrc, pdst)
    logits = _tc_out(z, dec_w2, dec_b2.reshape(1, NCLS))
    return logits[:P]


def kernel(x, edge_index, edge_type, pairs, W1, root1, b1, W2, root2, b2,
           dec_w1, dec_b1, dec_w2, dec_b2):
    return _run(x, edge_index, edge_type, pairs, W1, root1, b1, W2, root2, b2,
                dec_w1, dec_b1, dec_w2, dec_b2)


# speedup vs baseline: 34.5987x; 1.0117x over previous
import jax
import jax.numpy as jnp
from jax import lax
from jax.experimental import pallas as pl
from jax.experimental.pallas import tpu as pltpu
from jax.experimental.pallas import tpu_sc as plsc

N = 10000
E = 320000
R = 8
F_IN = 128
HID = 256
F_OUT = 128
NCLS = 16
P = 100000
P_PAD = 102400

NC = 2
NS = 16


BE = 80


def _make_sc_seg():
    ept = E // (NC * NS)
    nb = ept // BE
    assert nb % 2 == 1 and nb >= 3

    def body(table, gix, d32, out, gidx0, gidx1, sidx0, sidx1,
             gall, dall, rows0, rows1, zbuf, destbuf, sem0, sem1):
        core = lax.axis_index("c")
        tid = lax.axis_index("s")
        base_e = (core * NS + tid) * ept

        @pl.loop(0, 40)
        def _(i):
            for j in range(F_OUT // 16):
                zbuf[i, pl.ds(j * 16, 16)] = jnp.zeros((16,), jnp.float32)

        zrows = 640

        @pl.loop(0, zrows // 40)
        def _(k):
            pltpu.sync_copy(zbuf,
                            destbuf.at[pl.ds(tid * zrows + k * 40, 40)])

        pltpu.sync_copy(gix.at[pl.ds(base_e, ept)], gall)
        pltpu.sync_copy(d32.at[pl.ds(base_e, ept)], dall)
        plsc.subcore_barrier()

        def fill(slotg, slots, b):
            @pl.loop(0, BE // 16)
            def _(i):
                s = pl.ds(i * 16, 16)
                t = pl.ds(b * BE + i * 16, 16)
                slotg[s] = gall[t]
                slots[s] = dall[t]

        fill(gidx0, sidx0, 0)
        pltpu.async_copy(table.at[gidx0], rows0, sem0)

        @pl.loop(0, (nb - 1) // 2)
        def _(p):
            fill(gidx1, sidx1, 2 * p + 1)
            pltpu.async_copy(table.at[gidx1], rows1, sem1)

            pltpu.make_async_copy(table.at[gidx0], rows0, sem0).wait()
            pltpu.sync_copy(rows0, destbuf.at[sidx0], add=True)

            fill(gidx0, sidx0, 2 * p + 2)
            pltpu.async_copy(table.at[gidx0], rows0, sem0)

            pltpu.make_async_copy(table.at[gidx1], rows1, sem1).wait()
            pltpu.sync_copy(rows1, destbuf.at[sidx1], add=True)

        pltpu.make_async_copy(table.at[gidx0], rows0, sem0).wait()
        pltpu.sync_copy(rows0, destbuf.at[sidx0], add=True)

        plsc.subcore_barrier()

        @pl.when(tid < 15)
        def _():
            pltpu.sync_copy(destbuf.at[pl.ds(tid * zrows, zrows)],
                            out.at[pl.ds(core * N + tid * zrows, zrows)])

        @pl.when(tid == 15)
        def _():
            pltpu.sync_copy(destbuf.at[pl.ds(15 * zrows, 400)],
                            out.at[pl.ds(core * N + 15 * zrows, 400)])

        plsc.subcore_barrier()

    mesh = plsc.VectorSubcoreMesh(core_axis_name="c", subcore_axis_name="s")

    def run(table, gix, d32):
        return pl.kernel(
            body,
            out_type=jax.ShapeDtypeStruct((NC * N, F_OUT), jnp.float32),
            mesh=mesh,
            scratch_types=[
                pltpu.VMEM((BE,), jnp.int32),
                pltpu.VMEM((BE,), jnp.int32),
                pltpu.VMEM((BE,), jnp.int32),
                pltpu.VMEM((BE,), jnp.int32),
                pltpu.VMEM((E // (NC * NS),), jnp.int32),
                pltpu.VMEM((E // (NC * NS),), jnp.int32),
                pltpu.VMEM((BE, F_OUT), jnp.float32),
                pltpu.VMEM((BE, F_OUT), jnp.float32),
                pltpu.VMEM((40, F_OUT), jnp.float32),
                pltpu.VMEM_SHARED((16 * 640, F_OUT), jnp.float32),
                pltpu.SemaphoreType.DMA,
                pltpu.SemaphoreType.DMA,
            ],
        )(table, gix, d32)

    return run


_sc_seg = _make_sc_seg()


PPT = P_PAD // (NC * NS)
B3 = 160
NB3 = PPT // B3


def _sc_dec_body(hAB, psrc, pdst, z, pall, qall, ia0, ib0, ia1, ib1,
                 u0, v0, u1, v1, sa0, sb0, sa1, sb1):
    core = lax.axis_index("c")
    tid = lax.axis_index("s")
    w = core * NS + tid
    base_p = w * PPT

    pltpu.sync_copy(psrc.at[pl.ds(base_p, PPT)], pall)
    pltpu.sync_copy(pdst.at[pl.ds(base_p, PPT)], qall)

    def fill(ia, ib, b):
        @pl.loop(0, B3 // 16)
        def _(i):
            s = pl.ds(i * 16, 16)
            t = pl.ds(b * B3 + i * 16, 16)
            ia[s] = pall[t]
            ib[s] = qall[t] + N

    def fire(ia, ib, u, v, sa, sb):
        pltpu.async_copy(hAB.at[ia], u, sa)
        pltpu.async_copy(hAB.at[ib], v, sb)

    def finish(ia, ib, u, v, sa, sb, b):
        pltpu.make_async_copy(hAB.at[ia], u, sa).wait()
        pltpu.make_async_copy(hAB.at[ib], v, sb).wait()

        @pl.loop(0, B3, unroll=4)
        def _(i):
            for j in range(F_OUT // 16):
                s = pl.ds(j * 16, 16)
                u[i, s] = jnp.maximum(u[i, s] + v[i, s], 0.0)

        pltpu.sync_copy(u, z.at[pl.ds(base_p + b * B3, B3)])

    fill(ia0, ib0, 0)
    fire(ia0, ib0, u0, v0, sa0, sb0)

    @pl.loop(0, NB3 // 2)
    def _(p):
        fill(ia1, ib1, 2 * p + 1)
        fire(ia1, ib1, u1, v1, sa1, sb1)
        finish(ia0, ib0, u0, v0, sa0, sb0, 2 * p)

        @pl.when(p < NB3 // 2 - 1)
        def _():
            fill(ia0, ib0, 2 * p + 2)
            fire(ia0, ib0, u0, v0, sa0, sb0)

        finish(ia1, ib1, u1, v1, sa1, sb1, 2 * p + 1)


def _sc_dec(hAB, psrc, pdst):
    mesh = plsc.VectorSubcoreMesh(core_axis_name="c", subcore_axis_name="s")
    return pl.kernel(
        _sc_dec_body,
        out_type=jax.ShapeDtypeStruct((P_PAD, F_OUT), jnp.float32),
        mesh=mesh,
        scratch_types=[
            pltpu.VMEM((PPT,), jnp.int32),
            pltpu.VMEM((PPT,), jnp.int32),
            pltpu.VMEM((B3,), jnp.int32),
            pltpu.VMEM((B3,), jnp.int32),
            pltpu.VMEM((B3,), jnp.int32),
            pltpu.VMEM((B3,), jnp.int32),
            pltpu.VMEM((B3, F_OUT), jnp.float32),
            pltpu.VMEM((B3, F_OUT), jnp.float32),
            pltpu.VMEM((B3, F_OUT), jnp.float32),
            pltpu.VMEM((B3, F_OUT), jnp.float32),
            pltpu.SemaphoreType.DMA,
            pltpu.SemaphoreType.DMA,
            pltpu.SemaphoreType.DMA,
            pltpu.SemaphoreType.DMA,
        ],
    )(hAB, psrc, pdst)


NBLK = 1000
NNB = N // NBLK


def _tc_eidx_body(ei_blk, et_blk, gix_blk, d_blk):
    gix_blk[...] = et_blk[...] * N + ei_blk[0] + N
    d_blk[...] = ei_blk[1]


def _tc_eidx(edge_index2, edge_type2):
    eb = 2500
    return pl.pallas_call(
        _tc_eidx_body,
        grid=(E // (eb * 128),),
        in_specs=[
            pl.BlockSpec((2, eb, 128), lambda i: (0, i, 0)),
            pl.BlockSpec((eb, 128), lambda i: (i, 0)),
        ],
        out_specs=[
            pl.BlockSpec((eb, 128), lambda i: (i, 0)),
            pl.BlockSpec((eb, 128), lambda i: (i, 0)),
        ],
        out_shape=[
            jax.ShapeDtypeStruct((E // 128, 128), jnp.int32),
            jax.ShapeDtypeStruct((E // 128, 128), jnp.int32),
        ],
    )(edge_index2, edge_type2)


def _tc_transform_body(h_blk, w_blk, b_blk, out_blk):
    r = pl.program_id(1)
    y = jnp.dot(h_blk[...], w_blk[0], preferred_element_type=jnp.float32)
    out_blk[...] = jnp.where(r == 0, y + b_blk[...], y)


def _tc_transform(h, Wall, bias):
    in_dim = h.shape[1]
    return pl.pallas_call(
        _tc_transform_body,
        grid=(NNB, R + 1),
        in_specs=[
            pl.BlockSpec((NBLK, in_dim), lambda nb, r: (nb, 0)),
            pl.BlockSpec((1, in_dim, 128), lambda nb, r: (r, 0, 0)),
            pl.BlockSpec((1, 128), lambda nb, r: (0, 0)),
        ],
        out_specs=pl.BlockSpec((NBLK, 128), lambda nb, r: (r * NNB + nb, 0)),
        out_shape=jax.ShapeDtypeStruct(((R + 1) * N, 128), jnp.float32),
    )(h, Wall, bias)


def _tc_h1l2_body(lo_blk, hi_blk, pa0, pa1, pb0, pb1, w_blk, b2_blk,
                  out_blk, h1s):
    r = pl.program_id(1)

    @pl.when(r == 0)
    def _():
        h1s[:, 0:128] = jnp.maximum(lo_blk[...] + pa0[...] + pa1[...], 0.0)
        h1s[:, 128:HID] = jnp.maximum(hi_blk[...] + pb0[...] + pb1[...], 0.0)

    y = jnp.dot(h1s[...], w_blk[0], preferred_element_type=jnp.float32)
    out_blk[...] = jnp.where(r == 0, y + b2_blk[...], y)


def _tc_h1l2(y1lo, y1hi, partsA, partsB, W2all, b2):
    blk = lambda f: pl.BlockSpec((NBLK, 128), f)
    return pl.pallas_call(
        _tc_h1l2_body,
        grid=(NNB, R + 1),
        in_specs=[
            blk(lambda nb, r: (nb, 0)),
            blk(lambda nb, r: (nb, 0)),
            blk(lambda nb, r: (nb, 0)),
            blk(lambda nb, r: (NNB + nb, 0)),
            blk(lambda nb, r: (nb, 0)),
            blk(lambda nb, r: (NNB + nb, 0)),
            pl.BlockSpec((1, HID, F_OUT), lambda nb, r: (r, 0, 0)),
            pl.BlockSpec((1, F_OUT), lambda nb, r: (0, 0)),
        ],
        out_specs=pl.BlockSpec((NBLK, F_OUT), lambda nb, r: (r * NNB + nb, 0)),
        out_shape=jax.ShapeDtypeStruct(((R + 1) * N, F_OUT), jnp.float32),
        scratch_shapes=[pltpu.VMEM((NBLK, HID), jnp.float32)],
    )(y1lo, y1hi, partsA, partsA, partsB, partsB, W2all, b2)


def _tc_dec_prep_body(root_blk, p0_blk, p1_blk, w_blk, b_blk, out_blk):
    g = pl.program_id(0)
    h2 = root_blk[...] + p0_blk[...] + p1_blk[...]
    y = jnp.dot(h2, w_blk[...], preferred_element_type=jnp.float32)
    out_blk[...] = jnp.where(g == 0, y + b_blk[...], y)


def _tc_dec_prep(y2all, parts, dec_w1, dec_b1):
    return pl.pallas_call(
        _tc_dec_prep_body,
        grid=(2, NNB),
        in_specs=[
            pl.BlockSpec((NBLK, F_OUT), lambda g, nb: (nb, 0)),
            pl.BlockSpec((NBLK, F_OUT), lambda g, nb: (nb, 0)),
            pl.BlockSpec((NBLK, F_OUT), lambda g, nb: (NNB + nb, 0)),
            pl.BlockSpec((F_OUT, F_OUT), lambda g, nb: (g, 0)),
            pl.BlockSpec((1, F_OUT), lambda g, nb: (0, 0)),
        ],
        out_specs=pl.BlockSpec((NBLK, F_OUT), lambda g, nb: (g * NNB + nb, 0)),
        out_shape=jax.ShapeDtypeStruct((2 * N, F_OUT), jnp.float32),
    )(y2all, parts, parts, dec_w1, dec_b1)


ZBLK = 1600


def _tc_out_body(z_blk, w_blk, b_blk, out_blk):
    out_blk[...] = (
        jnp.dot(z_blk[...], w_blk[...], preferred_element_type=jnp.float32)
        + b_blk[...]
    )


def _tc_out(z, dec_w2, dec_b2):
    return pl.pallas_call(
        _tc_out_body,
        grid=(P_PAD // ZBLK,),
        in_specs=[
            pl.BlockSpec((ZBLK, F_OUT), lambda i: (i, 0)),
            pl.BlockSpec((F_OUT, NCLS), lambda i: (0, 0)),
            pl.BlockSpec((1, NCLS), lambda i: (0, 0)),
        ],
        out_specs=pl.BlockSpec((ZBLK, NCLS), lambda i: (i, 0)),
        out_shape=jax.ShapeDtypeStruct((P_PAD, NCLS), jnp.float32),
    )(z, dec_w2, dec_b2)


@jax.jit
def _run(x, edge_index, edge_type, pairs, W1, root1, b1, W2, root2, b2,
         dec_w1, dec_b1, dec_w2, dec_b2):
    ei2 = jnp.asarray(edge_index, jnp.int32).reshape(2, E // 128, 128)
    et2 = jnp.asarray(edge_type, jnp.int32).reshape(E // 128, 128)
    gix2, d2 = _tc_eidx(ei2, et2)
    gix = gix2.reshape(E)
    d32 = d2.reshape(E)
    pairs32 = jnp.asarray(pairs, jnp.int32)
    pairs_p = jnp.concatenate(
        [pairs32, jnp.zeros((P_PAD - P, 2), jnp.int32)], axis=0)
    psrc = pairs_p[:, 0]
    pdst = pairs_p[:, 1]

    root1r = root1.reshape(1, F_IN, HID)
    W1lo = jnp.concatenate([root1r[:, :, :128], W1[:, :, :128]], axis=0)
    W1hi = jnp.concatenate([root1r[:, :, 128:], W1[:, :, 128:]], axis=0)
    y1lo = _tc_transform(x, W1lo, b1[:128].reshape(1, 128))
    y1hi = _tc_transform(x, W1hi, b1[128:].reshape(1, 128))
    partsA = _sc_seg(y1lo, gix, d32)
    y1hi2, _ = lax.optimization_barrier((y1hi, partsA))
    partsB = _sc_seg(y1hi2, gix, d32)
    W2all = jnp.concatenate([root2.reshape(1, HID, F_OUT), W2], axis=0)
    y2all = _tc_h1l2(y1lo, y1hi2, partsA, partsB, W2all, b2.reshape(1, F_OUT))
    parts2 = _sc_seg(y2all, gix, d32)

    hAB = _tc_dec_prep(y2all, parts2, dec_w1, dec_b1.reshape(1, F_OUT))
    z = _sc_dec(hAB, psrc, pdst)
    logits = _tc_out(z, dec_w2, dec_b2.reshape(1, NCLS))
    return logits[:P]


def kernel(x, edge_index, edge_type, pairs, W1, root1, b1, W2, root2, b2,
           dec_w1, dec_b1, dec_w2, dec_b2):
    return _run(x, edge_index, edge_type, pairs, W1, root1, b1, W2, root2, b2,
                dec_w1, dec_b1, dec_w2, dec_b2)
